# Initial kernel scaffold; baseline (speedup 1.0000x reference)
import functools

import jax
import jax.numpy as jnp
from jax import lax
from jax.experimental import pallas as pl
from jax.experimental.pallas import tpu as pltpu
from jax.experimental.pallas import tpu_sc as plsc

BATCH = 4096
SEQ = 200
N_TOKENS = 100
EMBED_DIM = 32
SEQ_OUT = N_TOKENS + SEQ

NUM_CORES = 2
NUM_SUBCORES = 16
NW = NUM_CORES * NUM_SUBCORES
B_PER_W = BATCH // NW
G = 4
ROUNDS = B_PER_W // G
CHUNK = 100
NCHUNK = SEQ // CHUNK


def _body(ids_hbm, table_hbm, learned_hbm, out_hbm, idx_v, stage_v, sem):
    wid = lax.axis_index("s") * NUM_CORES + lax.axis_index("c")
    base = wid * B_PER_W

    for i in range(G):
        pltpu.sync_copy(learned_hbm, stage_v.at[i, pl.ds(0, N_TOKENS)])

    def round_body(r, carry):
        b0 = base + r * G
        pltpu.sync_copy(ids_hbm.at[pl.ds(b0, G)], idx_v)
        copies = []
        for i in range(G):
            for j in range(NCHUNK):
                copies.append(pltpu.async_copy(
                    table_hbm.at[idx_v.at[i, j]],
                    stage_v.at[i, pl.ds(N_TOKENS + j * CHUNK, CHUNK)],
                    sem))
        for cp in copies:
            cp.wait()
        pltpu.sync_copy(stage_v, out_hbm.at[pl.ds(b0, G)])
        return carry

    lax.fori_loop(0, ROUNDS, round_body, 0)


@jax.jit
def _soft_embedding(ids3, wte_weight, learned_embedding):
    mesh = plsc.VectorSubcoreMesh(core_axis_name="c", subcore_axis_name="s",
                                  num_cores=NUM_CORES,
                                  num_subcores=NUM_SUBCORES)
    fn = functools.partial(
        pl.kernel,
        out_type=jax.ShapeDtypeStruct((BATCH, SEQ_OUT, EMBED_DIM),
                                      jnp.float32),
        mesh=mesh,
        scratch_types=[
            pltpu.VMEM((G, NCHUNK, CHUNK), jnp.int32),
            pltpu.VMEM((G, SEQ_OUT, EMBED_DIM), jnp.float32),
            pltpu.SemaphoreType.DMA,
        ],
    )(_body)
    return fn(ids3, wte_weight, learned_embedding)


def kernel(input_ids, wte_weight, learned_embedding):
    ids3 = input_ids.astype(jnp.int32).reshape(BATCH, NCHUNK, CHUNK)
    return _soft_embedding(ids3, wte_weight, learned_embedding)


# speedup vs baseline: 1.3702x; 1.3702x over previous
import functools

import jax
import jax.numpy as jnp
from jax import lax
from jax.experimental import pallas as pl
from jax.experimental.pallas import tpu as pltpu
from jax.experimental.pallas import tpu_sc as plsc

BATCH = 4096
SEQ = 200
N_TOKENS = 100
EMBED_DIM = 32
SEQ_OUT = N_TOKENS + SEQ

NUM_CORES = 2
NUM_SUBCORES = 16
NW = NUM_CORES * NUM_SUBCORES
B_PER_W = BATCH // NW
G = 4
ROUNDS = B_PER_W // G
CHUNK = 100
NCHUNK = SEQ // CHUNK


def _body(ids_hbm, table_hbm, learned_hbm, out_hbm, idx_v, stage_v, sem):
    wid = lax.axis_index("s") * NUM_CORES + lax.axis_index("c")
    base = wid * B_PER_W

    for i in range(G):
        pltpu.sync_copy(learned_hbm, stage_v.at[i, pl.ds(0, N_TOKENS)])

    def round_body(r, carry):
        b0 = base + r * G
        pltpu.sync_copy(ids_hbm.at[pl.ds(b0, G)], idx_v)
        copies = []
        for i in range(G):
            for j in range(NCHUNK):
                copies.append(pltpu.async_copy(
                    table_hbm.at[idx_v.at[i, j]],
                    stage_v.at[i, pl.ds(N_TOKENS + j * CHUNK, CHUNK)],
                    sem))
        for cp in copies:
            cp.wait()
        pltpu.sync_copy(stage_v, out_hbm.at[pl.ds(b0, G)])
        return carry

    lax.fori_loop(0, ROUNDS, round_body, 0)


@jax.jit
def _soft_embedding(ids3, wte_weight, learned_embedding):
    mesh = plsc.VectorSubcoreMesh(core_axis_name="c", subcore_axis_name="s",
                                  num_cores=NUM_CORES,
                                  num_subcores=NUM_SUBCORES)
    fn = functools.partial(
        pl.kernel,
        out_type=jax.ShapeDtypeStruct((BATCH, SEQ_OUT, EMBED_DIM),
                                      jnp.float32),
        mesh=mesh,
        scratch_types=[
            pltpu.VMEM((G, NCHUNK, CHUNK), jnp.int32),
            pltpu.VMEM((G, SEQ_OUT, EMBED_DIM), jnp.float32),
            pltpu.SemaphoreType.DMA,
        ],
        compiler_params=pltpu.CompilerParams(use_tc_tiling_on_sc=False),
    )(_body)
    return fn(ids3, wte_weight, learned_embedding)


def kernel(input_ids, wte_weight, learned_embedding):
    ids3 = input_ids.astype(jnp.int32).reshape(BATCH, NCHUNK, CHUNK)
    return _soft_embedding(ids3, wte_weight, learned_embedding)


# speedup vs baseline: 1.4043x; 1.0249x over previous
import functools

import jax
import jax.numpy as jnp
from jax import lax
from jax.experimental import pallas as pl
from jax.experimental.pallas import tpu as pltpu
from jax.experimental.pallas import tpu_sc as plsc

BATCH = 4096
SEQ = 200
N_TOKENS = 100
EMBED_DIM = 32
SEQ_OUT = N_TOKENS + SEQ

NUM_CORES = 2
NUM_SUBCORES = 16
NW = NUM_CORES * NUM_SUBCORES
B_PER_W = BATCH // NW
G = 4
ROUNDS = B_PER_W // G
CHUNK = 100
NCHUNK = SEQ // CHUNK


def _body(ids_hbm, table_hbm, learned_hbm, out_hbm, idx_v, stage_v,
          sem0, sem1):
    wid = lax.axis_index("s") * NUM_CORES + lax.axis_index("c")
    base = wid * B_PER_W
    sems = (sem0, sem1)

    for buf in range(2):
        for i in range(G):
            pltpu.sync_copy(learned_hbm,
                            stage_v.at[buf, i, pl.ds(0, N_TOKENS)])

    def fire(buf, r):
        b0 = base + r * G
        pltpu.sync_copy(ids_hbm.at[pl.ds(b0, G)], idx_v.at[buf])
        for i in range(G):
            for j in range(NCHUNK):
                pltpu.async_copy(
                    table_hbm.at[idx_v.at[buf, i, j]],
                    stage_v.at[buf, i, pl.ds(N_TOKENS + j * CHUNK, CHUNK)],
                    sems[buf])

    def drain(buf):
        for i in range(G):
            for j in range(NCHUNK):
                pltpu.make_async_copy(
                    table_hbm.at[idx_v.at[buf, i, j]],
                    stage_v.at[buf, i, pl.ds(N_TOKENS + j * CHUNK, CHUNK)],
                    sems[buf]).wait()

    fire(0, 0)

    def outer(rr, carry):
        for b in range(2):
            r = rr * 2 + b

            @pl.when(r + 1 < ROUNDS)
            def _():
                fire(1 - b, r + 1)

            drain(b)
            pltpu.sync_copy(stage_v.at[b], out_hbm.at[pl.ds(base + r * G, G)])
        return carry

    lax.fori_loop(0, ROUNDS // 2, outer, 0)


@jax.jit
def _soft_embedding(ids3, wte_weight, learned_embedding):
    mesh = plsc.VectorSubcoreMesh(core_axis_name="c", subcore_axis_name="s",
                                  num_cores=NUM_CORES,
                                  num_subcores=NUM_SUBCORES)
    fn = functools.partial(
        pl.kernel,
        out_type=jax.ShapeDtypeStruct((BATCH, SEQ_OUT, EMBED_DIM),
                                      jnp.float32),
        mesh=mesh,
        scratch_types=[
            pltpu.VMEM((2, G, NCHUNK, CHUNK), jnp.int32),
            pltpu.VMEM((2, G, SEQ_OUT, EMBED_DIM), jnp.float32),
            pltpu.SemaphoreType.DMA,
            pltpu.SemaphoreType.DMA,
        ],
        compiler_params=pltpu.CompilerParams(use_tc_tiling_on_sc=False),
    )(_body)
    return fn(ids3, wte_weight, learned_embedding)


def kernel(input_ids, wte_weight, learned_embedding):
    ids3 = input_ids.astype(jnp.int32).reshape(BATCH, NCHUNK, CHUNK)
    return _soft_embedding(ids3, wte_weight, learned_embedding)
